# baseline (speedup 1.0000x reference)
import functools

import jax
import jax.numpy as jnp
from jax import lax
from jax.experimental import pallas as pl
from jax.experimental.pallas import tpu as pltpu
from jax.experimental.pallas import tpu_sc as plsc

V = 100000
D = 64
B = 4096
F = 26

NC = 2
NS = 16
NW = NC * NS

LANES = 128
SUB = 8
TCOLS = V // LANES
TAIL = V - TCOLS * LANES
CPW = 25
CPW_LAST = TCOLS - 31 * CPW

_mesh = plsc.VectorSubcoreMesh(
    core_axis_name="c", subcore_axis_name="s", num_cores=NC, num_subcores=NS
)


@functools.partial(
    pl.kernel,
    out_type=jax.ShapeDtypeStruct((V * D,), jnp.float32),
    mesh=_mesh,
    scratch_types=[
        pltpu.VMEM((D, LANES), jnp.float32),
        pltpu.VMEM((LANES * D,), jnp.float32),
        pltpu.VMEM((TAIL * D,), jnp.float32),
    ],
    compiler_params=pltpu.CompilerParams(
        use_tc_tiling_on_sc=True, needs_layout_passes=False
    ),
)
def _relayout(tt_hbm, tail_hbm, out_hbm, abuf, obuf, tbuf):
    wid = lax.axis_index("s") * NC + lax.axis_index("c")
    iota = lax.iota(jnp.int32, 16)
    d_idx = [iota + g * 16 for g in range(D // 16)]

    def do_col(c):
        pltpu.sync_copy(tt_hbm.at[:, pl.ds(c * LANES, LANES)], abuf)

        def xpose(jm, _):
            j_idx = jnp.broadcast_to(jm, (16,))
            for g in range(D // 16):
                vals = plsc.load_gather(abuf, [d_idx[g], j_idx])
                obuf[pl.ds(jm * D + g * 16, 16)] = vals
            return 0

        lax.fori_loop(0, LANES, xpose, 0)

    def full_cols(i, _):
        c = wid * CPW + i
        do_col(c)
        pltpu.sync_copy(obuf, out_hbm.at[pl.ds(c * LANES * D, LANES * D)])
        return 0

    ncols = jnp.where(wid == NW - 1, CPW_LAST, CPW)
    lax.fori_loop(0, ncols, full_cols, 0)

    @pl.when(wid == NW - 1)
    def _tail():
        pltpu.sync_copy(tail_hbm, tbuf)
        pltpu.sync_copy(tbuf, out_hbm.at[pl.ds(TCOLS * LANES * D, TAIL * D)])


BPW = B // NW
MC = 8
MB = BPW // MC
ROWS = MB * F


@functools.partial(
    pl.kernel,
    out_type=jax.ShapeDtypeStruct((B, D), jnp.float32),
    mesh=_mesh,
    scratch_types=[
        pltpu.VMEM((BPW, F), jnp.int32),
        pltpu.VMEM((ROWS, D), jnp.float32),
        pltpu.VMEM((ROWS, D), jnp.float32),
        pltpu.VMEM((BPW, D), jnp.float32),
        pltpu.SemaphoreType.DMA,
    ],
    compiler_params=pltpu.CompilerParams(use_tc_tiling_on_sc=False),
)
def _field_embed(x_hbm, table_hbm, out_hbm, idx_v, buf0, buf1, out_v, sem):
    wid = lax.axis_index("s") * NC + lax.axis_index("c")
    pltpu.sync_copy(x_hbm.at[pl.ds(wid * BPW, BPW)], idx_v)

    bufs = (buf0, buf1)

    def start_gather(m, buf):
        return [
            pltpu.async_copy(
                table_hbm.at[idx_v.at[m * MB + j]],
                buf.at[pl.ds(j * F, F)],
                sem,
            )
            for j in range(MB)
        ]

    copies = start_gather(0, bufs[0])
    for m in range(MC):
        buf = bufs[m % 2]
        for cp in copies:
            cp.wait()
        if m + 1 < MC:
            copies = start_gather(m + 1, bufs[(m + 1) % 2])

        def pool_row(b, _, buf=buf, m=m):
            base = b * F
            acc = [buf[base, pl.ds(d * 16, 16)] for d in range(D // 16)]
            for f in range(1, F):
                for d in range(D // 16):
                    acc[d] = acc[d] + buf[base + f, pl.ds(d * 16, 16)]
            row = m * MB + b
            for d in range(D // 16):
                out_v[row, pl.ds(d * 16, 16)] = acc[d]
            return 0

        lax.fori_loop(0, MB, pool_row, 0)

    pltpu.sync_copy(out_v, out_hbm.at[pl.ds(wid * BPW, BPW)])


def kernel(x, table):
    tail = table[TCOLS * LANES :, :].reshape(-1)
    t_flat = _relayout(table.T, tail)
    t2d = t_flat.reshape(V, D)
    return _field_embed(x.astype(jnp.int32), t2d)


# speedup vs baseline: 1.7460x; 1.7460x over previous
import functools

import jax
import jax.numpy as jnp
from jax import lax
from jax.experimental import pallas as pl
from jax.experimental.pallas import tpu as pltpu
from jax.experimental.pallas import tpu_sc as plsc

V = 100000
D = 64
B = 4096
F = 26

NC = 2
NS = 16
NW = NC * NS

LANES = 128
TCOLS = V // LANES
TAIL = V - TCOLS * LANES
KMAX = (TCOLS + NW - 1) // NW
WFULL = TCOLS - (KMAX - 1) * NW

_mesh = plsc.VectorSubcoreMesh(
    core_axis_name="c", subcore_axis_name="s", num_cores=NC, num_subcores=NS
)


@functools.partial(
    pl.kernel,
    out_type=jax.ShapeDtypeStruct((V * D,), jnp.float32),
    mesh=_mesh,
    scratch_types=[
        pltpu.VMEM((D, LANES), jnp.float32),
        pltpu.VMEM((D, LANES), jnp.float32),
        pltpu.VMEM((LANES * D,), jnp.float32),
        pltpu.VMEM((LANES * D,), jnp.float32),
        pltpu.VMEM((TAIL * D,), jnp.float32),
        pltpu.SemaphoreType.DMA,
        pltpu.SemaphoreType.DMA,
    ],
    compiler_params=pltpu.CompilerParams(
        use_tc_tiling_on_sc=True, needs_layout_passes=False
    ),
)
def _relayout(tt_hbm, tail_hbm, out_hbm, a0, a1, o0, o1, tbuf, sem_i, sem_o):
    wid = lax.axis_index("s") * NC + lax.axis_index("c")
    iota = lax.iota(jnp.int32, 16)
    d_idx = [iota + g * 16 for g in range(D // 16)]
    abufs, obufs = (a0, a1), (o0, o1)

    def col_of(k):
        return jnp.minimum(wid + k * NW, TCOLS - 1)

    def start_in(k):
        return pltpu.async_copy(
            tt_hbm.at[:, pl.ds(col_of(k) * LANES, LANES)], abufs[k % 2], sem_i
        )

    in_cp = [start_in(0), None]
    out_cp = [None, None]
    for k in range(KMAX):
        abuf, obuf = abufs[k % 2], obufs[k % 2]
        in_cp[k % 2].wait()
        if k + 1 < KMAX:
            in_cp[(k + 1) % 2] = start_in(k + 1)
        if out_cp[k % 2] is not None:
            out_cp[k % 2].wait()

        @plsc.parallel_loop(0, LANES, step=1, unroll=8)
        def xpose(jm, abuf=abuf, obuf=obuf):
            j_idx = jnp.broadcast_to(jm, (16,))
            for g in range(D // 16):
                obuf[pl.ds(jm * D + g * 16, 16)] = plsc.load_gather(
                    abuf, [d_idx[g], j_idx]
                )

        out_cp[k % 2] = pltpu.async_copy(
            obuf, out_hbm.at[pl.ds(col_of(k) * LANES * D, LANES * D)], sem_o
        )

    out_cp[(KMAX - 2) % 2].wait()
    out_cp[(KMAX - 1) % 2].wait()

    @pl.when(wid == NW - 1)
    def _tail():
        pltpu.sync_copy(tail_hbm, tbuf)
        pltpu.sync_copy(tbuf, out_hbm.at[pl.ds(TCOLS * LANES * D, TAIL * D)])


BPW = B // NW
MC = 8
MB = BPW // MC
ROWS = MB * F


@functools.partial(
    pl.kernel,
    out_type=jax.ShapeDtypeStruct((B, D), jnp.float32),
    mesh=_mesh,
    scratch_types=[
        pltpu.VMEM((BPW, F), jnp.int32),
        pltpu.VMEM((ROWS, D), jnp.float32),
        pltpu.VMEM((ROWS, D), jnp.float32),
        pltpu.VMEM((BPW, D), jnp.float32),
        pltpu.SemaphoreType.DMA,
    ],
    compiler_params=pltpu.CompilerParams(use_tc_tiling_on_sc=False),
)
def _field_embed(x_hbm, table_hbm, out_hbm, idx_v, buf0, buf1, out_v, sem):
    wid = lax.axis_index("s") * NC + lax.axis_index("c")
    pltpu.sync_copy(x_hbm.at[pl.ds(wid * BPW, BPW)], idx_v)

    bufs = (buf0, buf1)

    def start_gather(m, buf):
        return [
            pltpu.async_copy(
                table_hbm.at[idx_v.at[m * MB + j]],
                buf.at[pl.ds(j * F, F)],
                sem,
            )
            for j in range(MB)
        ]

    copies = start_gather(0, bufs[0])
    for m in range(MC):
        buf = bufs[m % 2]
        for cp in copies:
            cp.wait()
        if m + 1 < MC:
            copies = start_gather(m + 1, bufs[(m + 1) % 2])

        def pool_row(b, _, buf=buf, m=m):
            base = b * F
            acc = [buf[base, pl.ds(d * 16, 16)] for d in range(D // 16)]
            for f in range(1, F):
                for d in range(D // 16):
                    acc[d] = acc[d] + buf[base + f, pl.ds(d * 16, 16)]
            row = m * MB + b
            for d in range(D // 16):
                out_v[row, pl.ds(d * 16, 16)] = acc[d]
            return 0

        lax.fori_loop(0, MB, pool_row, 0)

    pltpu.sync_copy(out_v, out_hbm.at[pl.ds(wid * BPW, BPW)])


def kernel(x, table):
    tail = table[TCOLS * LANES :, :].reshape(-1)
    t_flat = _relayout(table.T, tail)
    t2d = t_flat.reshape(V, D)
    return _field_embed(x.astype(jnp.int32), t2d)
